# baseline (speedup 1.0000x reference)
import functools

import jax
import jax.numpy as jnp
from jax import lax
from jax.experimental import pallas as pl
from jax.experimental.pallas import tpu as pltpu
from jax.experimental.pallas import tpu_sc as plsc

N_USERS = 1000000
N_ITEMS = 1000000
D = 64
B = 16384

NC = 2
NS = 16
NW = NC * NS
BPW = B // NW
CHUNK = 32
NCH = BPW // CHUNK
NBROW = (N_USERS + 127) // 128


@functools.partial(
    pl.kernel,
    out_type=jax.ShapeDtypeStruct((B,), jnp.float32),
    mesh=plsc.VectorSubcoreMesh(core_axis_name="c", subcore_axis_name="s"),
    compiler_params=pltpu.CompilerParams(
        needs_layout_passes=False, use_tc_tiling_on_sc=True),
    scratch_types=[
        pltpu.VMEM((BPW,), jnp.int32),
        pltpu.VMEM((BPW,), jnp.int32),
        pltpu.VMEM((BPW,), jnp.int32),
        pltpu.VMEM((BPW,), jnp.int32),
        pltpu.VMEM((BPW,), jnp.int32),
        pltpu.VMEM((BPW,), jnp.int32),
        pltpu.VMEM((CHUNK, 128), jnp.float32),
        pltpu.VMEM((CHUNK, 128), jnp.float32),
        pltpu.VMEM((CHUNK, 128), jnp.float32),
        pltpu.VMEM((CHUNK, 128), jnp.float32),
        pltpu.VMEM((BPW,), jnp.float32),
        pltpu.SemaphoreType.DMA,
    ],
)
def _lfm_sc(users_h, items_h, ue2, ie2, ubp, ibp, out,
            uidx_v, iidx_v, urow_v, irow_v, ubr_v, ibr_v,
            ug_v, ig_v, ubg_v, ibg_v, out_v, sem):
    wid = lax.axis_index("s") * NC + lax.axis_index("c")
    base = wid * BPW

    pltpu.sync_copy(users_h.at[pl.ds(base, BPW)], uidx_v)
    pltpu.sync_copy(items_h.at[pl.ds(base, BPW)], iidx_v)

    def derive(j, carry):
        sl = pl.ds(j * 16, 16)
        uv = uidx_v[sl]
        iv = iidx_v[sl]
        urow_v[sl] = lax.shift_right_logical(uv, 1)
        irow_v[sl] = lax.shift_right_logical(iv, 1)
        ubr_v[sl] = lax.shift_right_logical(uv, 7)
        ibr_v[sl] = lax.shift_right_logical(iv, 7)
        return carry

    lax.fori_loop(0, BPW // 16, derive, 0)

    lane = lax.iota(jnp.int32, 16)

    def chunk_body(c, carry):
        cbase = c * CHUNK
        csl = pl.ds(cbase, CHUNK)
        cps = [
            pltpu.async_copy(ue2.at[urow_v.at[csl]], ug_v, sem),
            pltpu.async_copy(ie2.at[irow_v.at[csl]], ig_v, sem),
            pltpu.async_copy(ubp.at[ubr_v.at[csl]], ubg_v, sem),
            pltpu.async_copy(ibp.at[ibr_v.at[csl]], ibg_v, sem),
        ]
        for cp in cps:
            cp.wait()

        for g in range(CHUNK // 16):
            sl = pl.ds(cbase + g * 16, 16)
            uvec = uidx_v[sl]
            ivec = iidx_v[sl]
            uoff = (uvec & 1) * 64
            ioff = (ivec & 1) * 64
            row16 = g * 16 + lane
            tot = plsc.load_gather(ubg_v, [row16, uvec & 127])
            tot = tot + plsc.load_gather(ibg_v, [row16, ivec & 127])
            for l in range(16):
                r = g * 16 + l
                uo = uoff[l]
                io = ioff[l]
                acc = (ug_v[r, pl.ds(uo, 16)] * ig_v[r, pl.ds(io, 16)]
                       + ug_v[r, pl.ds(uo + 16, 16)] * ig_v[r, pl.ds(io + 16, 16)])
                acc = acc + (ug_v[r, pl.ds(uo + 32, 16)] * ig_v[r, pl.ds(io + 32, 16)]
                             + ug_v[r, pl.ds(uo + 48, 16)] * ig_v[r, pl.ds(io + 48, 16)])
                tot = jnp.where(lane == l, tot + jnp.sum(acc), tot)
            out_v[sl] = tot
        return carry

    lax.fori_loop(0, NCH, chunk_body, 0)

    pltpu.sync_copy(out_v, out.at[pl.ds(base, BPW)])


def kernel(users, items, user_embeddings, item_embeddings, user_biases, item_biases):
    ue2 = user_embeddings.reshape(N_USERS // 2, 2 * D)
    ie2 = item_embeddings.reshape(N_ITEMS // 2, 2 * D)
    ubp = jnp.pad(user_biases.reshape(N_USERS),
                  (0, NBROW * 128 - N_USERS)).reshape(NBROW, 128)
    ibp = jnp.pad(item_biases.reshape(N_ITEMS),
                  (0, NBROW * 128 - N_ITEMS)).reshape(NBROW, 128)
    return _lfm_sc(users.astype(jnp.int32), items.astype(jnp.int32),
                   ue2, ie2, ubp, ibp)
